# baseline (speedup 1.0000x reference)
import jax
import jax.numpy as jnp
from jax import lax
from jax.experimental import pallas as pl
from jax.experimental.pallas import tpu as pltpu
from jax.experimental.pallas import tpu_sc as plsc

_N_T = 200
_N_J = 25
_D_IN = 3
_D_M = 128
_B = 128
_S = _N_T * _N_J
_NC = 2
_NS = 16
_NW = _NC * _NS
_B_PER_W = _B // _NW
_C = 200
_N_CHUNK = _S // _C
_NV = _D_M // 16


def _sc_body(x_hbm, tt_hbm, jt_hbm, wt_hbm, b_hbm, nan_hbm, out_hbm,
             tbuf, jbuf, wbuf, bbuf, nanbuf, bnbuf, xbuf, obuf):
    wid = lax.axis_index("s") * _NC + lax.axis_index("c")
    pltpu.sync_copy(tt_hbm, tbuf)
    pltpu.sync_copy(jt_hbm, jbuf)
    pltpu.sync_copy(wt_hbm, wbuf)
    pltpu.sync_copy(b_hbm, bbuf)
    pltpu.sync_copy(nan_hbm, nanbuf)
    for v in range(_NV):
        sl = pl.ds(16 * v, 16)
        bnbuf[0, sl] = bbuf[0, sl] + nanbuf[0, sl]
        bnbuf[1, sl] = nanbuf[1, sl] - nanbuf[0, sl]

    def batch_body(bi, carry):
        bb = wid * _B_PER_W + bi
        pltpu.sync_copy(x_hbm.at[wid, bi, 0, :], xbuf.at[pl.ds(0, _S * _D_IN)])

        def chunk_body(ci, carry2):
            s0 = pl.multiple_of(ci * _C, _C)

            def row_body(r, carry3):
                sg = s0 + r
                xv = xbuf[pl.ds(_D_IN * sg, 16)]
                x0 = xv[0]
                x1 = xv[1]
                x2 = xv[2]
                m0 = x0 != x0
                m1 = x1 != x1
                m2 = x2 != x2
                mf = jnp.where(m0 | m1 | m2, 1.0, 0.0)
                xc0 = jnp.where(m0, 0.0, x0)
                xc1 = jnp.where(m1, 0.0, x1)
                xc2 = jnp.where(m2, 0.0, x2)
                tl = sg // _N_J
                jl = sg - tl * _N_J
                for v in range(_NV):
                    sl = pl.ds(16 * v, 16)
                    acc = tbuf[tl, sl] + jbuf[jl, sl] + bnbuf[0, sl]
                    acc = acc + mf * bnbuf[1, sl]
                    acc = acc + xc0 * wbuf[0, sl]
                    acc = acc + xc1 * wbuf[1, sl]
                    acc = acc + xc2 * wbuf[2, sl]
                    obuf[r, sl] = acc
                return carry3

            lax.fori_loop(0, _C, row_body, 0)
            pltpu.sync_copy(obuf, out_hbm.at[bb, pl.ds(s0, _C)])
            return carry2

        lax.fori_loop(0, _N_CHUNK, chunk_body, 0)
        return carry

    lax.fori_loop(0, _B_PER_W, batch_body, 0)


def kernel(x, W, b, time_table, joint_table, nan_table):
    xview = x.reshape(_NW, _B_PER_W, 1, _S * _D_IN)
    wt = W.T
    b2 = b.reshape(1, _D_M)
    mesh = plsc.VectorSubcoreMesh(core_axis_name="c", subcore_axis_name="s")
    f = pl.kernel(
        _sc_body,
        out_type=jax.ShapeDtypeStruct((_B, _S, _D_M), jnp.float32),
        mesh=mesh,
        scratch_types=[
            pltpu.VMEM((_N_T, _D_M), jnp.float32),
            pltpu.VMEM((_N_J, _D_M), jnp.float32),
            pltpu.VMEM((_D_IN, _D_M), jnp.float32),
            pltpu.VMEM((1, _D_M), jnp.float32),
            pltpu.VMEM((2, _D_M), jnp.float32),
            pltpu.VMEM((2, _D_M), jnp.float32),
            pltpu.VMEM((_S * _D_IN + 16, ), jnp.float32),
            pltpu.VMEM((_C, _D_M), jnp.float32),
        ],
    )
    return f(xview, time_table, joint_table, wt, b2, nan_table)


# speedup vs baseline: 1.1202x; 1.1202x over previous
import jax
import jax.numpy as jnp
from jax import lax
from jax.experimental import pallas as pl
from jax.experimental.pallas import tpu as pltpu
from jax.experimental.pallas import tpu_sc as plsc

_N_T = 200
_N_J = 25
_D_IN = 3
_D_M = 128
_B = 128
_S = _N_T * _N_J
_NC = 2
_NS = 16
_NW = _NC * _NS
_B_PER_W = _B // _NW
_C = 200
_T_C = _C // _N_J
_N_CHUNK = _S // _C
_NV = _D_M // 16


def _sc_body(x_hbm, tt_hbm, jt_hbm, wt_hbm, b_hbm, nan_hbm, out_hbm,
             tbuf, jbuf, wbuf, bbuf, nanbuf, jbase, xbuf, obuf):
    wid = lax.axis_index("s") * _NC + lax.axis_index("c")
    pltpu.sync_copy(tt_hbm, tbuf)
    pltpu.sync_copy(jt_hbm, jbuf)
    pltpu.sync_copy(wt_hbm, wbuf)
    pltpu.sync_copy(b_hbm, bbuf)
    pltpu.sync_copy(nan_hbm, nanbuf)

    def jbase_body(j, carry):
        for v in range(_NV):
            sl = pl.ds(16 * v, 16)
            jbase[j, sl] = jbuf[j, sl] + bbuf[0, sl] + nanbuf[0, sl]
        return carry

    lax.fori_loop(0, _N_J, jbase_body, 0)

    wv = [[wbuf[c, pl.ds(16 * v, 16)] for v in range(_NV)] for c in range(_D_IN)]
    dv = [nanbuf[1, pl.ds(16 * v, 16)] - nanbuf[0, pl.ds(16 * v, 16)]
          for v in range(_NV)]

    def batch_body(bi, carry):
        bb = wid * _B_PER_W + bi
        pltpu.sync_copy(x_hbm.at[wid, bi, 0, :], xbuf.at[pl.ds(0, _S * _D_IN)])

        def chunk_body(ci, carry2):
            s0 = pl.multiple_of(ci * _C, _C)

            def t_body(tl, carry3):
                tg = ci * _T_C + tl
                tv = [tbuf[tg, pl.ds(16 * v, 16)] for v in range(_NV)]
                row0 = tl * _N_J
                xoff0 = _D_IN * (s0 + row0)

                def j_body(j, carry4):
                    xv = xbuf[pl.ds(xoff0 + _D_IN * j, 16)]
                    x0 = xv[0]
                    x1 = xv[1]
                    x2 = xv[2]
                    m0 = x0 != x0
                    m1 = x1 != x1
                    m2 = x2 != x2
                    mf = jnp.where(m0 | m1 | m2, 1.0, 0.0)
                    xc0 = jnp.where(m0, 0.0, x0)
                    xc1 = jnp.where(m1, 0.0, x1)
                    xc2 = jnp.where(m2, 0.0, x2)
                    row = row0 + j
                    for v in range(_NV):
                        sl = pl.ds(16 * v, 16)
                        acc = tv[v] + jbase[j, sl]
                        acc = acc + xc0 * wv[0][v]
                        acc = acc + xc1 * wv[1][v]
                        acc = acc + xc2 * wv[2][v]
                        acc = acc + mf * dv[v]
                        obuf[row, sl] = acc
                    return carry4

                lax.fori_loop(0, _N_J, j_body, 0)
                return carry3

            lax.fori_loop(0, _T_C, t_body, 0)
            pltpu.sync_copy(obuf, out_hbm.at[bb, pl.ds(s0, _C)])
            return carry2

        lax.fori_loop(0, _N_CHUNK, chunk_body, 0)
        return carry

    lax.fori_loop(0, _B_PER_W, batch_body, 0)


def kernel(x, W, b, time_table, joint_table, nan_table):
    xview = x.reshape(_NW, _B_PER_W, 1, _S * _D_IN)
    wt = W.T
    b2 = b.reshape(1, _D_M)
    mesh = plsc.VectorSubcoreMesh(core_axis_name="c", subcore_axis_name="s")
    f = pl.kernel(
        _sc_body,
        out_type=jax.ShapeDtypeStruct((_B, _S, _D_M), jnp.float32),
        mesh=mesh,
        scratch_types=[
            pltpu.VMEM((_N_T, _D_M), jnp.float32),
            pltpu.VMEM((_N_J, _D_M), jnp.float32),
            pltpu.VMEM((_D_IN, _D_M), jnp.float32),
            pltpu.VMEM((1, _D_M), jnp.float32),
            pltpu.VMEM((2, _D_M), jnp.float32),
            pltpu.VMEM((_N_J, _D_M), jnp.float32),
            pltpu.VMEM((_S * _D_IN + 16,), jnp.float32),
            pltpu.VMEM((_C, _D_M), jnp.float32),
        ],
    )
    return f(xview, time_table, joint_table, wt, b2, nan_table)


# speedup vs baseline: 1.1742x; 1.0482x over previous
import jax
import jax.numpy as jnp
from jax import lax
from jax.experimental import pallas as pl
from jax.experimental.pallas import tpu as pltpu
from jax.experimental.pallas import tpu_sc as plsc

_N_T = 200
_N_J = 25
_D_IN = 3
_D_M = 128
_B = 128
_S = _N_T * _N_J
_NC = 2
_NS = 16
_NW = _NC * _NS
_B_PER_W = _B // _NW
_C = 200
_T_C = _C // _N_J
_N_CHUNK = _S // _C
_NV = _D_M // 16


def _sc_body(x_hbm, tt_hbm, jt_hbm, wt_hbm, b_hbm, nan_hbm, out_hbm,
             tbuf, jbuf, wbuf, bbuf, nanbuf, jbase, xbuf, obufA, obufB,
             semA, semB):
    wid = lax.axis_index("s") * _NC + lax.axis_index("c")
    pltpu.sync_copy(tt_hbm, tbuf)
    pltpu.sync_copy(jt_hbm, jbuf)
    pltpu.sync_copy(wt_hbm, wbuf)
    pltpu.sync_copy(b_hbm, bbuf)
    pltpu.sync_copy(nan_hbm, nanbuf)

    def jbase_body(j, carry):
        for v in range(_NV):
            sl = pl.ds(16 * v, 16)
            jbase[j, sl] = jbuf[j, sl] + bbuf[0, sl] + nanbuf[0, sl]
        return carry

    lax.fori_loop(0, _N_J, jbase_body, 0)

    wv = [[wbuf[c, pl.ds(16 * v, 16)] for v in range(_NV)] for c in range(_D_IN)]
    dv = [nanbuf[1, pl.ds(16 * v, 16)] - nanbuf[0, pl.ds(16 * v, 16)]
          for v in range(_NV)]

    def fill_chunk(ci, ob):
        s0 = pl.multiple_of(ci * _C, _C)

        def t_body(tl, carry3):
            tg = ci * _T_C + tl
            tv = [tbuf[tg, pl.ds(16 * v, 16)] for v in range(_NV)]
            row0 = tl * _N_J
            xoff0 = _D_IN * (s0 + row0)

            def j_body(j, carry4):
                xv = xbuf[pl.ds(xoff0 + _D_IN * j, 16)]
                x0 = xv[0]
                x1 = xv[1]
                x2 = xv[2]
                m0 = x0 != x0
                m1 = x1 != x1
                m2 = x2 != x2
                mf = jnp.where(m0 | m1 | m2, 1.0, 0.0)
                xc0 = jnp.where(m0, 0.0, x0)
                xc1 = jnp.where(m1, 0.0, x1)
                xc2 = jnp.where(m2, 0.0, x2)
                row = row0 + j
                for v in range(_NV):
                    sl = pl.ds(16 * v, 16)
                    acc = tv[v] + jbase[j, sl]
                    acc = acc + xc0 * wv[0][v]
                    acc = acc + xc1 * wv[1][v]
                    acc = acc + xc2 * wv[2][v]
                    acc = acc + mf * dv[v]
                    ob[row, sl] = acc
                return carry4

            lax.fori_loop(0, _N_J, j_body, 0, unroll=5)
            return carry3

        lax.fori_loop(0, _T_C, t_body, 0)

    def batch_body(bi, carry):
        bb = wid * _B_PER_W + bi
        pltpu.sync_copy(x_hbm.at[wid, bi, 0, :], xbuf.at[pl.ds(0, _S * _D_IN)])

        def drain(ob, sem):
            pltpu.make_async_copy(ob, out_hbm.at[bb, pl.ds(0, _C)], sem).wait()

        def pair_body(pi, carry2):
            for ob, sem, off in ((obufA, semA, 0), (obufB, semB, 1)):
                ci = 2 * pi + off

                @pl.when(pi > 0)
                def _():
                    drain(ob, sem)

                fill_chunk(ci, ob)
                s0 = pl.multiple_of(ci * _C, _C)
                pltpu.async_copy(ob, out_hbm.at[bb, pl.ds(s0, _C)], sem)
            return carry2

        lax.fori_loop(0, (_N_CHUNK - 1) // 2, pair_body, 0)
        drain(obufA, semA)
        fill_chunk(_N_CHUNK - 1, obufA)
        pltpu.async_copy(
            obufA, out_hbm.at[bb, pl.ds((_N_CHUNK - 1) * _C, _C)], semA)
        drain(obufA, semA)
        drain(obufB, semB)
        return carry

    lax.fori_loop(0, _B_PER_W, batch_body, 0)


def kernel(x, W, b, time_table, joint_table, nan_table):
    xview = x.reshape(_NW, _B_PER_W, 1, _S * _D_IN)
    wt = W.T
    b2 = b.reshape(1, _D_M)
    mesh = plsc.VectorSubcoreMesh(core_axis_name="c", subcore_axis_name="s")
    f = pl.kernel(
        _sc_body,
        out_type=jax.ShapeDtypeStruct((_B, _S, _D_M), jnp.float32),
        mesh=mesh,
        scratch_types=[
            pltpu.VMEM((_N_T, _D_M), jnp.float32),
            pltpu.VMEM((_N_J, _D_M), jnp.float32),
            pltpu.VMEM((_D_IN, _D_M), jnp.float32),
            pltpu.VMEM((1, _D_M), jnp.float32),
            pltpu.VMEM((2, _D_M), jnp.float32),
            pltpu.VMEM((_N_J, _D_M), jnp.float32),
            pltpu.VMEM((_S * _D_IN + 16,), jnp.float32),
            pltpu.VMEM((_C, _D_M), jnp.float32),
            pltpu.VMEM((_C, _D_M), jnp.float32),
            pltpu.SemaphoreType.DMA,
            pltpu.SemaphoreType.DMA,
        ],
    )
    return f(xview, time_table, joint_table, wt, b2, nan_table)


# speedup vs baseline: 2.4034x; 2.0469x over previous
import jax
import jax.numpy as jnp
from jax import lax
from jax.experimental import pallas as pl
from jax.experimental.pallas import tpu as pltpu
from jax.experimental.pallas import tpu_sc as plsc

_N_T = 200
_N_J = 25
_D_IN = 3
_D_M = 128
_B = 128
_S = _N_T * _N_J
_NC = 2
_NS = 16
_NW = _NC * _NS
_B_PER_W = _B // _NW
_C = 200
_T_C = _C // _N_J
_N_CHUNK = _S // _C
_NV = _D_M // 16


def _sc_body(x_hbm, tt_hbm, jt_hbm, wt_hbm, b_hbm, nan_hbm, out_hbm,
             tbuf, jbuf, wbuf, bbuf, nanbuf, jbase, xbuf, obufA, obufB,
             semA, semB):
    wid = lax.axis_index("s") * _NC + lax.axis_index("c")
    pltpu.sync_copy(tt_hbm, tbuf)
    pltpu.sync_copy(jt_hbm, jbuf)
    pltpu.sync_copy(wt_hbm, wbuf)
    pltpu.sync_copy(b_hbm, bbuf)
    pltpu.sync_copy(nan_hbm, nanbuf)

    def jbase_body(j, carry):
        for v in range(_NV):
            sl = pl.ds(16 * v, 16)
            jbase[j, sl] = jbuf[j, sl] + bbuf[0, sl] + nanbuf[0, sl]
        return carry

    lax.fori_loop(0, _N_J, jbase_body, 0)

    wv = [[wbuf[c, pl.ds(16 * v, 16)] for v in range(_NV)] for c in range(_D_IN)]
    dv = [nanbuf[1, pl.ds(16 * v, 16)] - nanbuf[0, pl.ds(16 * v, 16)]
          for v in range(_NV)]

    def fill_chunk(ci, ob):
        s0 = pl.multiple_of(ci * _C, _C)

        def t_body(tl, carry3):
            tg = ci * _T_C + tl
            tv = [tbuf[tg, pl.ds(16 * v, 16)] for v in range(_NV)]
            row0 = tl * _N_J
            xoff0 = _D_IN * (s0 + row0)

            def j_body(j, carry4):
                xv = xbuf[pl.ds(xoff0 + _D_IN * j, 16)]
                xi = lax.bitcast_convert_type(xv, jnp.int32)
                nanv = (xi & jnp.int32(0x7FFFFFFF)) > jnp.int32(0x7F800000)
                xc = jnp.where(nanv, 0.0, xv)
                nf = jnp.where(nanv, 1.0, 0.0)
                x0 = xc[0]
                x1 = xc[1]
                x2 = xc[2]
                mf = jnp.minimum(nf[0] + nf[1] + nf[2], 1.0)
                row = row0 + j
                sls = [pl.ds(16 * v, 16) for v in range(_NV)]
                acc = [jbase[j, sls[v]] + tv[v] for v in range(_NV)]
                p = [x0 * wv[0][v] for v in range(_NV)]
                acc = [acc[v] + p[v] for v in range(_NV)]
                p = [x1 * wv[1][v] for v in range(_NV)]
                acc = [acc[v] + p[v] for v in range(_NV)]
                p = [x2 * wv[2][v] for v in range(_NV)]
                acc = [acc[v] + p[v] for v in range(_NV)]
                p = [mf * dv[v] for v in range(_NV)]
                acc = [acc[v] + p[v] for v in range(_NV)]
                for v in range(_NV):
                    ob[row, sls[v]] = acc[v]
                return carry4

            lax.fori_loop(0, _N_J, j_body, 0, unroll=5)
            return carry3

        lax.fori_loop(0, _T_C, t_body, 0)

    def batch_body(bi, carry):
        bb = wid * _B_PER_W + bi
        pltpu.sync_copy(x_hbm.at[wid, bi, 0, :], xbuf.at[pl.ds(0, _S * _D_IN)])

        def drain(ob, sem):
            pltpu.make_async_copy(ob, out_hbm.at[bb, pl.ds(0, _C)], sem).wait()

        def pair_body(pi, carry2):
            for ob, sem, off in ((obufA, semA, 0), (obufB, semB, 1)):
                ci = 2 * pi + off

                @pl.when(pi > 0)
                def _():
                    drain(ob, sem)

                fill_chunk(ci, ob)
                s0 = pl.multiple_of(ci * _C, _C)
                pltpu.async_copy(ob, out_hbm.at[bb, pl.ds(s0, _C)], sem)
            return carry2

        lax.fori_loop(0, (_N_CHUNK - 1) // 2, pair_body, 0)
        drain(obufA, semA)
        fill_chunk(_N_CHUNK - 1, obufA)
        pltpu.async_copy(
            obufA, out_hbm.at[bb, pl.ds((_N_CHUNK - 1) * _C, _C)], semA)
        drain(obufA, semA)
        drain(obufB, semB)
        return carry

    lax.fori_loop(0, _B_PER_W, batch_body, 0)


def kernel(x, W, b, time_table, joint_table, nan_table):
    xview = x.reshape(_NW, _B_PER_W, 1, _S * _D_IN)
    wt = W.T
    b2 = b.reshape(1, _D_M)
    mesh = plsc.VectorSubcoreMesh(core_axis_name="c", subcore_axis_name="s")
    f = pl.kernel(
        _sc_body,
        out_type=jax.ShapeDtypeStruct((_B, _S, _D_M), jnp.float32),
        mesh=mesh,
        scratch_types=[
            pltpu.VMEM((_N_T, _D_M), jnp.float32),
            pltpu.VMEM((_N_J, _D_M), jnp.float32),
            pltpu.VMEM((_D_IN, _D_M), jnp.float32),
            pltpu.VMEM((1, _D_M), jnp.float32),
            pltpu.VMEM((2, _D_M), jnp.float32),
            pltpu.VMEM((_N_J, _D_M), jnp.float32),
            pltpu.VMEM((_S * _D_IN + 16,), jnp.float32),
            pltpu.VMEM((_C, _D_M), jnp.float32),
            pltpu.VMEM((_C, _D_M), jnp.float32),
            pltpu.SemaphoreType.DMA,
            pltpu.SemaphoreType.DMA,
        ],
    )
    return f(xview, time_table, joint_table, wt, b2, nan_table)


# speedup vs baseline: 4.0425x; 1.6820x over previous
import jax
import jax.numpy as jnp
from jax import lax
from jax.experimental import pallas as pl
from jax.experimental.pallas import tpu as pltpu
from jax.experimental.pallas import tpu_sc as plsc

_N_T = 200
_N_J = 25
_D_IN = 3
_D_M = 128
_B = 128
_S = _N_T * _N_J
_S_PAD = 5120
_NC = 2
_NS = 16
_NW = _NC * _NS
_B_PER_W = _B // _NW
_C = 200
_T_C = _C // _N_J
_N_CHUNK = _S // _C
_NV = _D_M // 16


def _sc_body(x0_hbm, x1_hbm, x2_hbm, tt_hbm, jt_hbm, wt_hbm, b_hbm, nan_hbm,
             out_hbm, tbuf, jbuf, wbuf, bbuf, nanbuf, jbase, xb0, xb1, xb2,
             obufA, obufB, semA, semB):
    wid = lax.axis_index("s") * _NC + lax.axis_index("c")
    pltpu.sync_copy(tt_hbm, tbuf)
    pltpu.sync_copy(jt_hbm, jbuf)
    pltpu.sync_copy(wt_hbm, wbuf)
    pltpu.sync_copy(b_hbm, bbuf)
    pltpu.sync_copy(nan_hbm, nanbuf)

    def jbase_body(j, carry):
        for v in range(_NV):
            sl = pl.ds(16 * v, 16)
            jbase[j, sl] = jbuf[j, sl] + bbuf[0, sl] + nanbuf[0, sl]
        return carry

    lax.fori_loop(0, _N_J, jbase_body, 0)

    wv = [[wbuf[c, pl.ds(16 * v, 16)] for v in range(_NV)] for c in range(_D_IN)]
    dv = [nanbuf[1, pl.ds(16 * v, 16)] - nanbuf[0, pl.ds(16 * v, 16)]
          for v in range(_NV)]

    def fill_chunk(ci, ob):
        s0 = pl.multiple_of(ci * _C, _C)

        def t_body(tl, carry3):
            tg = ci * _T_C + tl
            tv = [tbuf[tg, pl.ds(16 * v, 16)] for v in range(_NV)]
            row0 = tl * _N_J

            def j_body(j, carry4):
                sg = s0 + row0 + j
                xcs = []
                nfs = []
                for c, xbc in enumerate((xb0, xb1, xb2)):
                    vc = xbc[pl.ds(sg, 16)]
                    bc = jnp.full((16,), vc[0], jnp.float32)
                    ic = lax.bitcast_convert_type(bc, jnp.int32)
                    nanc = (ic & jnp.int32(0x7FFFFFFF)) > jnp.int32(0x7F800000)
                    xcs.append(jnp.where(nanc, 0.0, bc))
                    nfs.append(jnp.where(nanc, 1.0, 0.0))
                x0 = xcs[0]
                x1 = xcs[1]
                x2 = xcs[2]
                mf = jnp.minimum(nfs[0] + nfs[1] + nfs[2], 1.0)
                row = row0 + j
                sls = [pl.ds(16 * v, 16) for v in range(_NV)]
                acc = [jbase[j, sls[v]] + tv[v] for v in range(_NV)]
                p = [x0 * wv[0][v] for v in range(_NV)]
                acc = [acc[v] + p[v] for v in range(_NV)]
                p = [x1 * wv[1][v] for v in range(_NV)]
                acc = [acc[v] + p[v] for v in range(_NV)]
                p = [x2 * wv[2][v] for v in range(_NV)]
                acc = [acc[v] + p[v] for v in range(_NV)]
                p = [mf * dv[v] for v in range(_NV)]
                acc = [acc[v] + p[v] for v in range(_NV)]
                for v in range(_NV):
                    ob[row, sls[v]] = acc[v]
                return carry4

            lax.fori_loop(0, _N_J, j_body, 0, unroll=5)
            return carry3

        lax.fori_loop(0, _T_C, t_body, 0)

    def batch_body(bi, carry):
        bb = wid * _B_PER_W + bi
        pltpu.sync_copy(x0_hbm.at[bb, 0, :], xb0)
        pltpu.sync_copy(x1_hbm.at[bb, 0, :], xb1)
        pltpu.sync_copy(x2_hbm.at[bb, 0, :], xb2)

        def drain(ob, sem):
            pltpu.make_async_copy(ob, out_hbm.at[bb, pl.ds(0, _C)], sem).wait()

        def pair_body(pi, carry2):
            for ob, sem, off in ((obufA, semA, 0), (obufB, semB, 1)):
                ci = 2 * pi + off

                @pl.when(pi > 0)
                def _():
                    drain(ob, sem)

                fill_chunk(ci, ob)
                s0 = pl.multiple_of(ci * _C, _C)
                pltpu.async_copy(ob, out_hbm.at[bb, pl.ds(s0, _C)], sem)
            return carry2

        lax.fori_loop(0, (_N_CHUNK - 1) // 2, pair_body, 0)
        drain(obufA, semA)
        fill_chunk(_N_CHUNK - 1, obufA)
        pltpu.async_copy(
            obufA, out_hbm.at[bb, pl.ds((_N_CHUNK - 1) * _C, _C)], semA)
        drain(obufA, semA)
        drain(obufB, semB)
        return carry

    lax.fori_loop(0, _B_PER_W, batch_body, 0)


def kernel(x, W, b, time_table, joint_table, nan_table):
    xt = jnp.pad(jnp.swapaxes(x, 1, 2), ((0, 0), (0, 0), (0, _S_PAD - _S)))
    xp = [xt[:, c:c + 1, :] for c in range(_D_IN)]
    wt = W.T
    b2 = b.reshape(1, _D_M)
    mesh = plsc.VectorSubcoreMesh(core_axis_name="c", subcore_axis_name="s")
    f = pl.kernel(
        _sc_body,
        out_type=jax.ShapeDtypeStruct((_B, _S, _D_M), jnp.float32),
        mesh=mesh,
        scratch_types=[
            pltpu.VMEM((_N_T, _D_M), jnp.float32),
            pltpu.VMEM((_N_J, _D_M), jnp.float32),
            pltpu.VMEM((_D_IN, _D_M), jnp.float32),
            pltpu.VMEM((1, _D_M), jnp.float32),
            pltpu.VMEM((2, _D_M), jnp.float32),
            pltpu.VMEM((_N_J, _D_M), jnp.float32),
            pltpu.VMEM((_S_PAD,), jnp.float32),
            pltpu.VMEM((_S_PAD,), jnp.float32),
            pltpu.VMEM((_S_PAD,), jnp.float32),
            pltpu.VMEM((_C, _D_M), jnp.float32),
            pltpu.VMEM((_C, _D_M), jnp.float32),
            pltpu.SemaphoreType.DMA,
            pltpu.SemaphoreType.DMA,
        ],
    )
    return f(xp[0], xp[1], xp[2], time_table, joint_table, wt, b2, nan_table)


# speedup vs baseline: 4.4256x; 1.0948x over previous
import jax
import jax.numpy as jnp
from jax import lax
from jax.experimental import pallas as pl
from jax.experimental.pallas import tpu as pltpu
from jax.experimental.pallas import tpu_sc as plsc

_N_T = 200
_N_J = 25
_D_IN = 3
_D_M = 128
_B = 128
_S = _N_T * _N_J
_S_PAD = 5120
_NC = 2
_NS = 16
_NW = _NC * _NS
_B_PER_W = _B // _NW
_C = 200
_T_C = _C // _N_J
_N_CHUNK = _S // _C
_NV = _D_M // 16


def _sc_body(x0_hbm, x1_hbm, x2_hbm, tt_hbm, jt_hbm, wt_hbm, b_hbm, nan_hbm,
             out_hbm, tbuf, jbuf, wbuf, bbuf, nanbuf, jbase, xb0, xb1, xb2,
             obufA, obufB, semA, semB):
    wid = lax.axis_index("s") * _NC + lax.axis_index("c")
    pltpu.sync_copy(tt_hbm, tbuf)
    pltpu.sync_copy(jt_hbm, jbuf)
    pltpu.sync_copy(wt_hbm, wbuf)
    pltpu.sync_copy(b_hbm, bbuf)
    pltpu.sync_copy(nan_hbm, nanbuf)

    def jbase_body(j, carry):
        for v in range(_NV):
            sl = pl.ds(16 * v, 16)
            jbase[j, sl] = jbuf[j, sl] + bbuf[0, sl] + nanbuf[0, sl]
        return carry

    lax.fori_loop(0, _N_J, jbase_body, 0)

    wv = [[wbuf[c, pl.ds(16 * v, 16)] for v in range(_NV)] for c in range(_D_IN)]
    dv = [nanbuf[1, pl.ds(16 * v, 16)] - nanbuf[0, pl.ds(16 * v, 16)]
          for v in range(_NV)]

    def fill_chunk(ci, ob):
        s0 = pl.multiple_of(ci * _C, _C)

        def t_body(tl, carry3):
            tg = ci * _T_C + tl
            tv = [tbuf[tg, pl.ds(16 * v, 16)] for v in range(_NV)]
            row0 = tl * _N_J
            base = s0 + row0
            xw = []
            nfw = []
            for xbc in (xb0, xb1, xb2):
                cl = []
                nf = []
                for off in (0, 16):
                    vwin = xbc[pl.ds(base + off, 16)]
                    iw = lax.bitcast_convert_type(vwin, jnp.int32)
                    nanw = ((iw & jnp.int32(0x7FFFFFFF))
                            > jnp.int32(0x7F800000))
                    cl.append(jnp.where(nanw, 0.0, vwin))
                    nf.append(jnp.where(nanw, 1.0, 0.0))
                xw.append(cl)
                nfw.append(nf)
            mfw = [jnp.minimum(nfw[0][w] + nfw[1][w] + nfw[2][w], 1.0)
                   for w in range(2)]
            sls = [pl.ds(16 * v, 16) for v in range(_NV)]
            for j in range(_N_J):
                w, l = divmod(j, 16)
                x0 = jnp.full((16,), xw[0][w][l], jnp.float32)
                x1 = jnp.full((16,), xw[1][w][l], jnp.float32)
                x2 = jnp.full((16,), xw[2][w][l], jnp.float32)
                mf = jnp.full((16,), mfw[w][l], jnp.float32)
                row = row0 + j
                acc = [jbase[j, sls[v]] + tv[v] for v in range(_NV)]
                p = [x0 * wv[0][v] for v in range(_NV)]
                acc = [acc[v] + p[v] for v in range(_NV)]
                p = [x1 * wv[1][v] for v in range(_NV)]
                acc = [acc[v] + p[v] for v in range(_NV)]
                p = [x2 * wv[2][v] for v in range(_NV)]
                acc = [acc[v] + p[v] for v in range(_NV)]
                p = [mf * dv[v] for v in range(_NV)]
                acc = [acc[v] + p[v] for v in range(_NV)]
                for v in range(_NV):
                    ob[row, sls[v]] = acc[v]
            return carry3

        lax.fori_loop(0, _T_C, t_body, 0)

    def batch_body(bi, carry):
        bb = wid * _B_PER_W + bi
        pltpu.sync_copy(x0_hbm.at[bb, 0, :], xb0)
        pltpu.sync_copy(x1_hbm.at[bb, 0, :], xb1)
        pltpu.sync_copy(x2_hbm.at[bb, 0, :], xb2)

        def drain(ob, sem):
            pltpu.make_async_copy(ob, out_hbm.at[bb, pl.ds(0, _C)], sem).wait()

        def pair_body(pi, carry2):
            for ob, sem, off in ((obufA, semA, 0), (obufB, semB, 1)):
                ci = 2 * pi + off

                @pl.when(pi > 0)
                def _():
                    drain(ob, sem)

                fill_chunk(ci, ob)
                s0 = pl.multiple_of(ci * _C, _C)
                pltpu.async_copy(ob, out_hbm.at[bb, pl.ds(s0, _C)], sem)
            return carry2

        lax.fori_loop(0, (_N_CHUNK - 1) // 2, pair_body, 0)
        drain(obufA, semA)
        fill_chunk(_N_CHUNK - 1, obufA)
        pltpu.async_copy(
            obufA, out_hbm.at[bb, pl.ds((_N_CHUNK - 1) * _C, _C)], semA)
        drain(obufA, semA)
        drain(obufB, semB)
        return carry

    lax.fori_loop(0, _B_PER_W, batch_body, 0)


def kernel(x, W, b, time_table, joint_table, nan_table):
    xt = jnp.pad(jnp.swapaxes(x, 1, 2), ((0, 0), (0, 0), (0, _S_PAD - _S)))
    xp = [xt[:, c:c + 1, :] for c in range(_D_IN)]
    wt = W.T
    b2 = b.reshape(1, _D_M)
    mesh = plsc.VectorSubcoreMesh(core_axis_name="c", subcore_axis_name="s")
    f = pl.kernel(
        _sc_body,
        out_type=jax.ShapeDtypeStruct((_B, _S, _D_M), jnp.float32),
        mesh=mesh,
        scratch_types=[
            pltpu.VMEM((_N_T, _D_M), jnp.float32),
            pltpu.VMEM((_N_J, _D_M), jnp.float32),
            pltpu.VMEM((_D_IN, _D_M), jnp.float32),
            pltpu.VMEM((1, _D_M), jnp.float32),
            pltpu.VMEM((2, _D_M), jnp.float32),
            pltpu.VMEM((_N_J, _D_M), jnp.float32),
            pltpu.VMEM((_S_PAD,), jnp.float32),
            pltpu.VMEM((_S_PAD,), jnp.float32),
            pltpu.VMEM((_S_PAD,), jnp.float32),
            pltpu.VMEM((_C, _D_M), jnp.float32),
            pltpu.VMEM((_C, _D_M), jnp.float32),
            pltpu.SemaphoreType.DMA,
            pltpu.SemaphoreType.DMA,
        ],
    )
    return f(xp[0], xp[1], xp[2], time_table, joint_table, wt, b2, nan_table)
